# Initial kernel scaffold; baseline (speedup 1.0000x reference)
import jax
import jax.numpy as jnp
from jax.experimental import pallas as pl


def kernel(x, edge_index, W1, b1, W2, b2):
    raise NotImplementedError("write your pallas kernel here")


# speedup vs baseline: 31.5201x; 31.5201x over previous
import jax
import jax.numpy as jnp
from jax import lax
from jax.experimental import pallas as pl
from jax.experimental.pallas import tpu as pltpu
from jax.experimental.pallas import tpu_sc as plsc

N = 10000
NP = 10112
E = 320000
F = 128
H = 16
C = 40
NSC = 2
NSUB = 16
NTILES = NSC * NSUB
CHUNK = 128
NCHUNK = 79
EPAD = NTILES * CHUNK * NCHUNK
RPT = NP // NSUB

_mesh = plsc.VectorSubcoreMesh(core_axis_name="c", subcore_axis_name="s")
_sc_params = pltpu.CompilerParams(use_tc_tiling_on_sc=False)


def _fill_rows(buf, nrows, value):
    def body(i, carry):
        buf[i, :] = jnp.full((H,), value, jnp.float32)
        return carry
    lax.fori_loop(0, nrows, body, 0)


def _deg_body(dst_hbm, out_hbm, dst_v, ones_v, zero_v, shared):
    c = lax.axis_index("c")
    s = lax.axis_index("s")
    g = c * NSUB + s
    _fill_rows(zero_v, RPT, 0.0)
    _fill_rows(ones_v, CHUNK, 1.0)
    pltpu.sync_copy(zero_v, shared.at[pl.ds(s * RPT, RPT)])
    plsc.subcore_barrier()
    pltpu.sync_copy(dst_hbm.at[g], dst_v)

    def body(j, carry):
        pltpu.sync_copy(ones_v, shared.at[dst_v.at[j]], add=True)
        return carry

    lax.fori_loop(0, NCHUNK, body, 0)
    plsc.subcore_barrier()
    pltpu.sync_copy(shared.at[pl.ds(s * RPT, RPT)],
                    out_hbm.at[c, pl.ds(s * RPT, RPT)])


def _agg_body(hs_hbm, src_hbm, dst_hbm, out_hbm,
              src_v, dst_v, rows_v, zero_v, sem, shared):
    c = lax.axis_index("c")
    s = lax.axis_index("s")
    g = c * NSUB + s
    _fill_rows(zero_v, RPT, 0.0)
    pltpu.sync_copy(zero_v, shared.at[pl.ds(s * RPT, RPT)])
    plsc.subcore_barrier()
    pltpu.sync_copy(src_hbm.at[g], src_v)
    pltpu.sync_copy(dst_hbm.at[g], dst_v)

    def body(j, carry):
        pltpu.async_copy(hs_hbm.at[src_v.at[j]], rows_v, sem).wait()
        pltpu.sync_copy(rows_v, shared.at[dst_v.at[j]], add=True)
        return carry

    lax.fori_loop(0, NCHUNK, body, 0)
    plsc.subcore_barrier()
    pltpu.sync_copy(shared.at[pl.ds(s * RPT, RPT)],
                    out_hbm.at[c, pl.ds(s * RPT, RPT)])


_deg_call = pl.kernel(
    _deg_body,
    out_type=jax.ShapeDtypeStruct((NSC, NP, H), jnp.float32),
    mesh=_mesh,
    scratch_types=[
        pltpu.VMEM((NCHUNK, CHUNK), jnp.int32),
        pltpu.VMEM((CHUNK, H), jnp.float32),
        pltpu.VMEM((RPT, H), jnp.float32),
        pltpu.VMEM_SHARED((NP, H), jnp.float32),
    ],
    compiler_params=_sc_params,
)

_agg_call = pl.kernel(
    _agg_body,
    out_type=jax.ShapeDtypeStruct((NSC, NP, H), jnp.float32),
    mesh=_mesh,
    scratch_types=[
        pltpu.VMEM((NCHUNK, CHUNK), jnp.int32),
        pltpu.VMEM((NCHUNK, CHUNK), jnp.int32),
        pltpu.VMEM((CHUNK, H), jnp.float32),
        pltpu.VMEM((RPT, H), jnp.float32),
        pltpu.SemaphoreType.DMA,
        pltpu.VMEM_SHARED((NP, H), jnp.float32),
    ],
    compiler_params=_sc_params,
)


def _tc1_body(x_ref, w1_ref, degp_ref, hs1_ref, dis_ref):
    deg = degp_ref[0] + degp_ref[1] + 1.0
    dis = lax.rsqrt(deg)[:N]
    h = jnp.dot(x_ref[...], w1_ref[...], preferred_element_type=jnp.float32)
    hs1_ref[...] = h * dis
    dis_ref[...] = dis


def _tc2_body(aggp_ref, hs1_ref, dis_ref, b1_ref, hs2_ref):
    agg = aggp_ref[0][:N] + aggp_ref[1][:N] + hs1_ref[...]
    t = agg * dis_ref[...] + b1_ref[...]
    r = jnp.maximum(t, 0.0)
    hs2_ref[...] = r * dis_ref[...]


def _tc3_body(aggp_ref, hs2_ref, dis_ref, w2_ref, b2_ref, out_ref):
    sagg = (aggp_ref[0][:N] + aggp_ref[1][:N] + hs2_ref[...]) * dis_ref[...]
    h2 = jnp.dot(sagg, w2_ref[...], preferred_element_type=jnp.float32)
    h2 = h2 + b2_ref[...]
    m = jnp.max(h2, axis=1, keepdims=True)
    lse = jnp.log(jnp.sum(jnp.exp(h2 - m), axis=1, keepdims=True)) + m
    out_ref[...] = h2 - lse


_tc1 = pl.pallas_call(
    _tc1_body,
    out_shape=[
        jax.ShapeDtypeStruct((N, H), jnp.float32),
        jax.ShapeDtypeStruct((N, H), jnp.float32),
    ],
)

_tc2 = pl.pallas_call(
    _tc2_body,
    out_shape=jax.ShapeDtypeStruct((N, H), jnp.float32),
)

_tc3 = pl.pallas_call(
    _tc3_body,
    out_shape=jax.ShapeDtypeStruct((N, C), jnp.float32),
)


def kernel(x, edge_index, W1, b1, W2, b2):
    src = edge_index[0].astype(jnp.int32)
    dst = edge_index[1].astype(jnp.int32)
    pad = EPAD - E
    src3 = jnp.concatenate(
        [src, jnp.zeros((pad,), jnp.int32)]).reshape(NTILES, NCHUNK, CHUNK)
    dst3 = jnp.concatenate(
        [dst, N + (jnp.arange(pad, dtype=jnp.int32) % NSUB)]
    ).reshape(NTILES, NCHUNK, CHUNK)

    degp = _deg_call(dst3)
    hs1, dis = _tc1(x, W1, degp)
    agg1 = _agg_call(hs1, src3, dst3)
    hs2 = _tc2(agg1, hs1, dis, b1.reshape(1, H))
    agg2 = _agg_call(hs2, src3, dst3)
    return _tc3(agg2, hs2, dis, W2, b2.reshape(1, C))


# speedup vs baseline: 56.0589x; 1.7785x over previous
import jax
import jax.numpy as jnp
from jax import lax
from jax.experimental import pallas as pl
from jax.experimental.pallas import tpu as pltpu
from jax.experimental.pallas import tpu_sc as plsc

N = 10000
NP = 10112
E = 320000
F = 128
H = 16
C = 40
NSC = 2
NSUB = 16
NTILES = NSC * NSUB
CHUNK = 128
NCHUNK = 80
SROWS = N // NSUB
EPAD = NTILES * CHUNK * NCHUNK
RPT = NP // NSUB

_mesh = plsc.VectorSubcoreMesh(core_axis_name="c", subcore_axis_name="s")
_sc_params = pltpu.CompilerParams(use_tc_tiling_on_sc=False)


def _fill_rows(buf, nrows, value):
    def body(i, carry):
        buf[i, :] = jnp.full((H,), value, jnp.float32)
        return carry
    lax.fori_loop(0, nrows, body, 0)


def _deg_body(dst_hbm, out_hbm, dst_v, ones_v, zero_v, shared):
    c = lax.axis_index("c")
    s = lax.axis_index("s")
    g = c * NSUB + s
    _fill_rows(zero_v, RPT, 0.0)
    _fill_rows(ones_v, CHUNK, 1.0)
    pltpu.sync_copy(zero_v, shared.at[pl.ds(s * RPT, RPT)])
    plsc.subcore_barrier()
    pltpu.sync_copy(dst_hbm.at[g], dst_v)

    def body(j, carry):
        pltpu.sync_copy(ones_v, shared.at[dst_v.at[j]], add=True)
        return carry

    lax.fori_loop(0, NCHUNK, body, 0)
    plsc.subcore_barrier()
    pltpu.sync_copy(shared.at[pl.ds(s * RPT, RPT)],
                    out_hbm.at[c, pl.ds(s * RPT, RPT)])


def _agg_body(hs_hbm, src_hbm, dst_hbm, out_hbm,
              src_v, dst_v, rows0, rows1, zero_v, sem0, sem1,
              hs_sh, shared):
    c = lax.axis_index("c")
    s = lax.axis_index("s")
    g = c * NSUB + s
    _fill_rows(zero_v, RPT, 0.0)
    pltpu.sync_copy(zero_v, shared.at[pl.ds(s * RPT, RPT)])
    pltpu.sync_copy(hs_hbm.at[pl.ds(s * SROWS, SROWS)],
                    hs_sh.at[pl.ds(s * SROWS, SROWS)])
    plsc.subcore_barrier()
    pltpu.sync_copy(src_hbm.at[g], src_v)
    pltpu.sync_copy(dst_hbm.at[g], dst_v)

    def gather(j, buf, sem):
        pltpu.async_copy(hs_sh.at[src_v.at[j]], buf, sem)

    def gwait(buf, sem):
        pltpu.make_async_copy(hs_sh.at[src_v.at[0]], buf, sem).wait()

    gather(0, rows0, sem0)
    gather(1, rows1, sem1)

    def body(i, carry):
        j = 2 * i
        gwait(rows0, sem0)
        pltpu.sync_copy(rows0, shared.at[dst_v.at[j]], add=True)
        gather(jnp.minimum(j + 2, NCHUNK - 1), rows0, sem0)
        gwait(rows1, sem1)
        pltpu.sync_copy(rows1, shared.at[dst_v.at[j + 1]], add=True)
        gather(jnp.minimum(j + 3, NCHUNK - 1), rows1, sem1)
        return carry

    lax.fori_loop(0, NCHUNK // 2, body, 0)
    gwait(rows0, sem0)
    gwait(rows1, sem1)
    plsc.subcore_barrier()
    pltpu.sync_copy(shared.at[pl.ds(s * RPT, RPT)],
                    out_hbm.at[c, pl.ds(s * RPT, RPT)])


_deg_call = pl.kernel(
    _deg_body,
    out_type=jax.ShapeDtypeStruct((NSC, NP, H), jnp.float32),
    mesh=_mesh,
    scratch_types=[
        pltpu.VMEM((NCHUNK, CHUNK), jnp.int32),
        pltpu.VMEM((CHUNK, H), jnp.float32),
        pltpu.VMEM((RPT, H), jnp.float32),
        pltpu.VMEM_SHARED((NP, H), jnp.float32),
    ],
    compiler_params=_sc_params,
)

_agg_call = pl.kernel(
    _agg_body,
    out_type=jax.ShapeDtypeStruct((NSC, NP, H), jnp.float32),
    mesh=_mesh,
    scratch_types=[
        pltpu.VMEM((NCHUNK, CHUNK), jnp.int32),
        pltpu.VMEM((NCHUNK, CHUNK), jnp.int32),
        pltpu.VMEM((CHUNK, H), jnp.float32),
        pltpu.VMEM((CHUNK, H), jnp.float32),
        pltpu.VMEM((RPT, H), jnp.float32),
        pltpu.SemaphoreType.DMA,
        pltpu.SemaphoreType.DMA,
        pltpu.VMEM_SHARED((N, H), jnp.float32),
        pltpu.VMEM_SHARED((NP, H), jnp.float32),
    ],
    compiler_params=_sc_params,
)


def _tc1_body(x_ref, w1_ref, degp_ref, hs1_ref, dis_ref):
    deg = degp_ref[0] + degp_ref[1] + 1.0
    dis = lax.rsqrt(deg)[:N]
    h = jnp.dot(x_ref[...], w1_ref[...], preferred_element_type=jnp.float32)
    hs1_ref[...] = h * dis
    dis_ref[...] = dis


def _tc2_body(aggp_ref, hs1_ref, dis_ref, b1_ref, hs2_ref):
    agg = aggp_ref[0][:N] + aggp_ref[1][:N] + hs1_ref[...]
    t = agg * dis_ref[...] + b1_ref[...]
    r = jnp.maximum(t, 0.0)
    hs2_ref[...] = r * dis_ref[...]


def _tc3_body(aggp_ref, hs2_ref, dis_ref, w2_ref, b2_ref, out_ref):
    sagg = (aggp_ref[0][:N] + aggp_ref[1][:N] + hs2_ref[...]) * dis_ref[...]
    h2 = jnp.dot(sagg, w2_ref[...], preferred_element_type=jnp.float32)
    h2 = h2 + b2_ref[...]
    m = jnp.max(h2, axis=1, keepdims=True)
    lse = jnp.log(jnp.sum(jnp.exp(h2 - m), axis=1, keepdims=True)) + m
    out_ref[...] = h2 - lse


_tc1 = pl.pallas_call(
    _tc1_body,
    out_shape=[
        jax.ShapeDtypeStruct((N, H), jnp.float32),
        jax.ShapeDtypeStruct((N, H), jnp.float32),
    ],
)

_tc2 = pl.pallas_call(
    _tc2_body,
    out_shape=jax.ShapeDtypeStruct((N, H), jnp.float32),
)

_tc3 = pl.pallas_call(
    _tc3_body,
    out_shape=jax.ShapeDtypeStruct((N, C), jnp.float32),
)


def kernel(x, edge_index, W1, b1, W2, b2):
    src = edge_index[0].astype(jnp.int32)
    dst = edge_index[1].astype(jnp.int32)
    pad = EPAD - E
    src3 = jnp.concatenate(
        [src, jnp.zeros((pad,), jnp.int32)]).reshape(NTILES, NCHUNK, CHUNK)
    dst3 = jnp.concatenate(
        [dst, N + (jnp.arange(pad, dtype=jnp.int32) % (NP - N))]
    ).reshape(NTILES, NCHUNK, CHUNK)

    degp = _deg_call(dst3)
    hs1, dis = _tc1(x, W1, degp)
    agg1 = _agg_call(hs1, src3, dst3)
    hs2 = _tc2(agg1, hs1, dis, b1.reshape(1, H))
    agg2 = _agg_call(hs2, src3, dst3)
    return _tc3(agg2, hs2, dis, W2, b2.reshape(1, C))


# speedup vs baseline: 59.7101x; 1.0651x over previous
import jax
import jax.numpy as jnp
from jax import lax
from jax.experimental import pallas as pl
from jax.experimental.pallas import tpu as pltpu
from jax.experimental.pallas import tpu_sc as plsc

N = 10000
NP = 10112
E = 320000
F = 128
H = 16
C = 40
NSC = 2
NSUB = 16
NTILES = NSC * NSUB
CHUNK = 128
NCHUNK = 80
SROWS = N // NSUB
EPAD = NTILES * CHUNK * NCHUNK
RPT = NP // NSUB

_mesh = plsc.VectorSubcoreMesh(core_axis_name="c", subcore_axis_name="s")
_sc_params = pltpu.CompilerParams(use_tc_tiling_on_sc=False)


def _fill_rows(buf, nrows, value):
    def body(i, carry):
        buf[i, :] = jnp.full((H,), value, jnp.float32)
        return carry
    lax.fori_loop(0, nrows, body, 0)


def _deg_body(dst_hbm, out_hbm, dst_v, ones_v, zero_v, sem, shared):
    c = lax.axis_index("c")
    s = lax.axis_index("s")
    g = c * NSUB + s
    _fill_rows(zero_v, RPT, 0.0)
    _fill_rows(ones_v, CHUNK, 1.0)
    pltpu.sync_copy(zero_v, shared.at[pl.ds(s * RPT, RPT)])
    plsc.subcore_barrier()
    pltpu.sync_copy(dst_hbm.at[g], dst_v)

    def body(gi, carry):
        for b in range(8):
            pltpu.async_copy(ones_v, shared.at[dst_v.at[gi * 8 + b]], sem,
                             add=True)
        for _ in range(8):
            pltpu.make_async_copy(ones_v, shared.at[dst_v.at[0]], sem).wait()
        return carry

    lax.fori_loop(0, NCHUNK // 8, body, 0)
    plsc.subcore_barrier()
    pltpu.sync_copy(shared.at[pl.ds(s * RPT, RPT)],
                    out_hbm.at[c, pl.ds(s * RPT, RPT)])


def _agg_tail(src_hbm, dst_hbm, out_hbm,
              src_v, dst_v, rows0, rows1, sem0, sem1, hs_sh, shared, c, s, g):
    pltpu.sync_copy(src_hbm.at[g], src_v)
    pltpu.sync_copy(dst_hbm.at[g], dst_v)

    def gather(j, buf, sem):
        pltpu.async_copy(hs_sh.at[src_v.at[j]], buf, sem)

    def gwait(buf, sem):
        pltpu.make_async_copy(hs_sh.at[src_v.at[0]], buf, sem).wait()

    gather(0, rows0, sem0)
    gather(1, rows1, sem1)

    def body(i, carry):
        j = 2 * i
        gwait(rows0, sem0)
        pltpu.sync_copy(rows0, shared.at[dst_v.at[j]], add=True)
        gather(jnp.minimum(j + 2, NCHUNK - 1), rows0, sem0)
        gwait(rows1, sem1)
        pltpu.sync_copy(rows1, shared.at[dst_v.at[j + 1]], add=True)
        gather(jnp.minimum(j + 3, NCHUNK - 1), rows1, sem1)
        return carry

    lax.fori_loop(0, NCHUNK // 2, body, 0)
    gwait(rows0, sem0)
    gwait(rows1, sem1)
    plsc.subcore_barrier()
    pltpu.sync_copy(shared.at[pl.ds(s * RPT, RPT)],
                    out_hbm.at[c, pl.ds(s * RPT, RPT)])


def _agg_body(hs_hbm, src_hbm, dst_hbm, out_hbm,
              src_v, dst_v, rows0, rows1, zero_v, sem0, sem1,
              hs_sh, shared):
    c = lax.axis_index("c")
    s = lax.axis_index("s")
    g = c * NSUB + s
    _fill_rows(zero_v, RPT, 0.0)
    pltpu.sync_copy(zero_v, shared.at[pl.ds(s * RPT, RPT)])
    pltpu.sync_copy(hs_hbm.at[pl.ds(s * SROWS, SROWS)],
                    hs_sh.at[pl.ds(s * SROWS, SROWS)])
    plsc.subcore_barrier()
    _agg_tail(src_hbm, dst_hbm, out_hbm, src_v, dst_v, rows0, rows1,
              sem0, sem1, hs_sh, shared, c, s, g)


def _agg2_body(aggp_hbm, hs1_hbm, dis_hbm, b1_hbm, src_hbm, dst_hbm,
               out_hbm, hs2_hbm,
               src_v, dst_v, rows0, rows1, zero_v, p0_v, p1_v, t_v, d_v, b1_v,
               sem0, sem1, hs_sh, shared):
    c = lax.axis_index("c")
    s = lax.axis_index("s")
    g = c * NSUB + s
    _fill_rows(zero_v, RPT, 0.0)
    pltpu.sync_copy(zero_v, shared.at[pl.ds(s * RPT, RPT)])
    r0 = pl.ds(s * SROWS, SROWS)
    pltpu.sync_copy(aggp_hbm.at[0, r0], p0_v)
    pltpu.sync_copy(aggp_hbm.at[1, r0], p1_v)
    pltpu.sync_copy(hs1_hbm.at[r0], t_v)
    pltpu.sync_copy(dis_hbm.at[r0], d_v)
    pltpu.sync_copy(b1_hbm, b1_v)
    b1v = b1_v[...]

    def sbody(i, carry):
        d = d_v[i, :]
        t = (p0_v[i, :] + p1_v[i, :] + t_v[i, :]) * d + b1v
        t_v[i, :] = jnp.maximum(t, 0.0) * d
        return carry

    lax.fori_loop(0, SROWS, sbody, 0, unroll=5)
    pltpu.sync_copy(t_v, hs_sh.at[r0])
    pltpu.sync_copy(t_v, hs2_hbm.at[r0])
    plsc.subcore_barrier()
    _agg_tail(src_hbm, dst_hbm, out_hbm, src_v, dst_v, rows0, rows1,
              sem0, sem1, hs_sh, shared, c, s, g)


_deg_call = pl.kernel(
    _deg_body,
    out_type=jax.ShapeDtypeStruct((NSC, NP, H), jnp.float32),
    mesh=_mesh,
    scratch_types=[
        pltpu.VMEM((NCHUNK, CHUNK), jnp.int32),
        pltpu.VMEM((CHUNK, H), jnp.float32),
        pltpu.VMEM((RPT, H), jnp.float32),
        pltpu.SemaphoreType.DMA,
        pltpu.VMEM_SHARED((NP, H), jnp.float32),
    ],
    compiler_params=_sc_params,
)

_agg_call = pl.kernel(
    _agg_body,
    out_type=jax.ShapeDtypeStruct((NSC, NP, H), jnp.float32),
    mesh=_mesh,
    scratch_types=[
        pltpu.VMEM((NCHUNK, CHUNK), jnp.int32),
        pltpu.VMEM((NCHUNK, CHUNK), jnp.int32),
        pltpu.VMEM((CHUNK, H), jnp.float32),
        pltpu.VMEM((CHUNK, H), jnp.float32),
        pltpu.VMEM((RPT, H), jnp.float32),
        pltpu.SemaphoreType.DMA,
        pltpu.SemaphoreType.DMA,
        pltpu.VMEM_SHARED((N, H), jnp.float32),
        pltpu.VMEM_SHARED((NP, H), jnp.float32),
    ],
    compiler_params=_sc_params,
)

_agg2_call = pl.kernel(
    _agg2_body,
    out_type=[
        jax.ShapeDtypeStruct((NSC, NP, H), jnp.float32),
        jax.ShapeDtypeStruct((N, H), jnp.float32),
    ],
    mesh=_mesh,
    scratch_types=[
        pltpu.VMEM((NCHUNK, CHUNK), jnp.int32),
        pltpu.VMEM((NCHUNK, CHUNK), jnp.int32),
        pltpu.VMEM((CHUNK, H), jnp.float32),
        pltpu.VMEM((CHUNK, H), jnp.float32),
        pltpu.VMEM((RPT, H), jnp.float32),
        pltpu.VMEM((SROWS, H), jnp.float32),
        pltpu.VMEM((SROWS, H), jnp.float32),
        pltpu.VMEM((SROWS, H), jnp.float32),
        pltpu.VMEM((SROWS, H), jnp.float32),
        pltpu.VMEM((H,), jnp.float32),
        pltpu.SemaphoreType.DMA,
        pltpu.SemaphoreType.DMA,
        pltpu.VMEM_SHARED((N, H), jnp.float32),
        pltpu.VMEM_SHARED((NP, H), jnp.float32),
    ],
    compiler_params=_sc_params,
)


def _tc1_body(x_ref, w1_ref, degp_ref, hs1_ref, dis_ref):
    deg = degp_ref[0] + degp_ref[1] + 1.0
    dis = lax.rsqrt(deg)[:N]
    h = jnp.dot(x_ref[...], w1_ref[...], preferred_element_type=jnp.float32)
    hs1_ref[...] = h * dis
    dis_ref[...] = dis


def _tc3_body(aggp_ref, hs2_ref, dis_ref, w2_ref, b2_ref, out_ref):
    sagg = (aggp_ref[0][:N] + aggp_ref[1][:N] + hs2_ref[...]) * dis_ref[...]
    h2 = jnp.dot(sagg, w2_ref[...], preferred_element_type=jnp.float32)
    h2 = h2 + b2_ref[...]
    m = jnp.max(h2, axis=1, keepdims=True)
    lse = jnp.log(jnp.sum(jnp.exp(h2 - m), axis=1, keepdims=True)) + m
    out_ref[...] = h2 - lse


_tc1 = pl.pallas_call(
    _tc1_body,
    out_shape=[
        jax.ShapeDtypeStruct((N, H), jnp.float32),
        jax.ShapeDtypeStruct((N, H), jnp.float32),
    ],
)

_tc3 = pl.pallas_call(
    _tc3_body,
    out_shape=jax.ShapeDtypeStruct((N, C), jnp.float32),
)


def kernel(x, edge_index, W1, b1, W2, b2):
    src = edge_index[0].astype(jnp.int32)
    dst = edge_index[1].astype(jnp.int32)
    pad = EPAD - E
    src3 = jnp.concatenate(
        [src, jnp.zeros((pad,), jnp.int32)]).reshape(NTILES, NCHUNK, CHUNK)
    dst3 = jnp.concatenate(
        [dst, N + (jnp.arange(pad, dtype=jnp.int32) % (NP - N))]
    ).reshape(NTILES, NCHUNK, CHUNK)

    degp = _deg_call(dst3)
    hs1, dis = _tc1(x, W1, degp)
    agg1 = _agg_call(hs1, src3, dst3)
    agg2, hs2 = _agg2_call(agg1, hs1, dis, b1, src3, dst3)
    return _tc3(agg2, hs2, dis, W2, b2.reshape(1, C))
